# Initial kernel scaffold; baseline (speedup 1.0000x reference)
import jax
import jax.numpy as jnp
from jax.experimental import pallas as pl


def kernel(z, means, devs, mix_partition):
    raise NotImplementedError("write your pallas kernel here")


# speedup vs baseline: 31.4508x; 31.4508x over previous
import functools

import jax
import jax.numpy as jnp
from jax import lax
from jax.experimental import pallas as pl
from jax.experimental.pallas import tpu as pltpu
from jax.experimental.pallas import tpu_sc as plsc

N = 1000000
D = 8
K = 1024
NW = 32
R = 1024
NFULL = N // R
TAIL = N - NFULL * R
TASKS = -(-NFULL // NW)
TAIL_W = 16


def _process_rows(zv, outv, devs_v, means_v, part_v, ngroups):
    lane = lax.iota(jnp.int32, 16)

    def group(g, _):
        rid = g * 16 + lane
        zoff = rid * (D + 1)
        u = plsc.load_gather(zv, [zoff])
        x = [plsc.load_gather(zv, [zoff + (1 + j)]) for j in range(D)]

        lo = jnp.zeros(16, jnp.int32)
        hi = jnp.full((16,), K, jnp.int32)
        for _step in range(11):
            mid = (lo + hi) // 2
            pm = plsc.load_gather(part_v, [jnp.minimum(mid, K - 1)])
            take_hi = pm <= u
            lo = jnp.where(take_hi, mid + 1, lo)
            hi = jnp.where(take_hi, hi, mid)
        idx = jnp.minimum(lo, K - 1)

        mbase = idx * D
        dbase = idx * (D * D)
        obase = rid * D
        for i in range(D):
            acc = plsc.load_gather(means_v, [mbase + i])
            for j in range(D):
                acc = acc + plsc.load_gather(devs_v, [dbase + (i * D + j)]) * x[j]
            plsc.store_scatter(outv, [obase + i], acc)
        return 0

    lax.fori_loop(0, ngroups, group, 0)


def _body(z_hbm, means_hbm, devs_hbm, part_hbm, out_hbm,
          devs_v, means_v, part_v, zv, outv):
    pltpu.sync_copy(devs_hbm, devs_v)
    pltpu.sync_copy(means_hbm, means_v)
    pltpu.sync_copy(part_hbm, part_v)

    wid = lax.axis_index("s") * 2 + lax.axis_index("c")

    def task(t, _):
        b = wid + NW * t

        @pl.when(b < NFULL)
        def _():
            zoff = pl.multiple_of(b * (R * (D + 1)), 8)
            ooff = pl.multiple_of(b * (R * D), 8)
            pltpu.sync_copy(z_hbm.at[pl.ds(zoff, R * (D + 1))], zv)
            _process_rows(zv, outv, devs_v, means_v, part_v, R // 16)
            pltpu.sync_copy(outv, out_hbm.at[pl.ds(ooff, R * D)])

        return 0

    lax.fori_loop(0, TASKS, task, 0)

    @pl.when(wid == TAIL_W)
    def _():
        zoff = pl.multiple_of(NFULL * (R * (D + 1)), 8)
        ooff = pl.multiple_of(NFULL * (R * D), 8)
        pltpu.sync_copy(z_hbm.at[pl.ds(zoff, TAIL * (D + 1))],
                        zv.at[pl.ds(0, TAIL * (D + 1))])
        _process_rows(zv, outv, devs_v, means_v, part_v, TAIL // 16)
        pltpu.sync_copy(outv.at[pl.ds(0, TAIL * D)],
                        out_hbm.at[pl.ds(ooff, TAIL * D)])


@jax.jit
def _run(zf, meansf, devsf, part):
    mesh = plsc.VectorSubcoreMesh(core_axis_name="c", subcore_axis_name="s")
    return pl.kernel(
        _body,
        mesh=mesh,
        compiler_params=pltpu.CompilerParams(needs_layout_passes=False),
        out_type=jax.ShapeDtypeStruct((N * D,), jnp.float32),
        scratch_types=[
            pltpu.VMEM((K * D * D,), jnp.float32),
            pltpu.VMEM((K * D,), jnp.float32),
            pltpu.VMEM((K,), jnp.float32),
            pltpu.VMEM((R * (D + 1),), jnp.float32),
            pltpu.VMEM((R * D,), jnp.float32),
        ],
    )(zf, meansf, devsf, part)


def kernel(z, means, devs, mix_partition):
    out = _run(z.reshape(-1), means.reshape(-1), devs.reshape(-1),
               mix_partition)
    return out.reshape(N, D)


# speedup vs baseline: 35.1954x; 1.1191x over previous
import functools

import jax
import jax.numpy as jnp
from jax import lax
from jax.experimental import pallas as pl
from jax.experimental.pallas import tpu as pltpu
from jax.experimental.pallas import tpu_sc as plsc

N = 1000000
D = 8
K = 1024
NW = 32
R = 1024
NFULL = N // R
TAIL = N - NFULL * R
TASKS = -(-NFULL // NW)
TAIL_W = 16


def _process_rows(zv, outv, devs_v, means_v, part_v, ngroups):
    lane = lax.iota(jnp.int32, 16)

    @plsc.parallel_loop(0, ngroups, unroll=4)
    def group(g):
        rid = g * 16 + lane
        zoff = rid * (D + 1)
        u = plsc.load_gather(zv, [zoff])
        x = [plsc.load_gather(zv, [zoff + (1 + j)]) for j in range(D)]

        lo = jnp.zeros(16, jnp.int32)
        hi = jnp.full((16,), K, jnp.int32)
        for _step in range(11):
            mid = (lo + hi) // 2
            pm = plsc.load_gather(part_v, [jnp.minimum(mid, K - 1)])
            take_hi = pm <= u
            lo = jnp.where(take_hi, mid + 1, lo)
            hi = jnp.where(take_hi, hi, mid)
        idx = jnp.minimum(lo, K - 1)

        mbase = idx * D
        dbase = idx * (D * D)
        obase = rid * D
        for i in range(D):
            acc = plsc.load_gather(means_v, [mbase + i])
            for j in range(D):
                acc = acc + plsc.load_gather(devs_v, [dbase + (i * D + j)]) * x[j]
            plsc.store_scatter(outv, [obase + i], acc)


def _body(z_hbm, means_hbm, devs_hbm, part_hbm, out_hbm,
          devs_v, means_v, part_v, zv, outv):
    pltpu.sync_copy(devs_hbm, devs_v)
    pltpu.sync_copy(means_hbm, means_v)
    pltpu.sync_copy(part_hbm, part_v)

    wid = lax.axis_index("s") * 2 + lax.axis_index("c")

    def task(t, _):
        b = wid + NW * t

        @pl.when(b < NFULL)
        def _():
            zoff = pl.multiple_of(b * (R * (D + 1)), 8)
            ooff = pl.multiple_of(b * (R * D), 8)
            pltpu.sync_copy(z_hbm.at[pl.ds(zoff, R * (D + 1))], zv)
            _process_rows(zv, outv, devs_v, means_v, part_v, R // 16)
            pltpu.sync_copy(outv, out_hbm.at[pl.ds(ooff, R * D)])

        return 0

    lax.fori_loop(0, TASKS, task, 0)

    @pl.when(wid == TAIL_W)
    def _():
        zoff = pl.multiple_of(NFULL * (R * (D + 1)), 8)
        ooff = pl.multiple_of(NFULL * (R * D), 8)
        pltpu.sync_copy(z_hbm.at[pl.ds(zoff, TAIL * (D + 1))],
                        zv.at[pl.ds(0, TAIL * (D + 1))])
        _process_rows(zv, outv, devs_v, means_v, part_v, TAIL // 16)
        pltpu.sync_copy(outv.at[pl.ds(0, TAIL * D)],
                        out_hbm.at[pl.ds(ooff, TAIL * D)])


@jax.jit
def _run(zf, meansf, devsf, part):
    mesh = plsc.VectorSubcoreMesh(core_axis_name="c", subcore_axis_name="s")
    return pl.kernel(
        _body,
        mesh=mesh,
        compiler_params=pltpu.CompilerParams(needs_layout_passes=False),
        out_type=jax.ShapeDtypeStruct((N * D,), jnp.float32),
        scratch_types=[
            pltpu.VMEM((K * D * D,), jnp.float32),
            pltpu.VMEM((K * D,), jnp.float32),
            pltpu.VMEM((K,), jnp.float32),
            pltpu.VMEM((R * (D + 1),), jnp.float32),
            pltpu.VMEM((R * D,), jnp.float32),
        ],
    )(zf, meansf, devsf, part)


def kernel(z, means, devs, mix_partition):
    out = _run(z.reshape(-1), means.reshape(-1), devs.reshape(-1),
               mix_partition)
    return out.reshape(N, D)


# speedup vs baseline: 53.7475x; 1.5271x over previous
import functools

import jax
import jax.numpy as jnp
from jax import lax
from jax.experimental import pallas as pl
from jax.experimental.pallas import tpu as pltpu
from jax.experimental.pallas import tpu_sc as plsc

N = 1000000
D = 8
K = 1024
NW = 32
R = 1024
NFULL = N // R
TAIL = N - NFULL * R
TASKS = -(-NFULL // NW)
TAIL_W = 16
DSTRIDE = D * D + 1
MSTRIDE = D + 1


def _process_rows(zv, outv, devs_v, means_v, part_v, ngroups):
    lane = lax.iota(jnp.int32, 16)

    @plsc.parallel_loop(0, ngroups, unroll=4)
    def group(g):
        rid = g * 16 + lane
        zoff = rid * (D + 1)
        u = plsc.load_gather(zv, [zoff])
        x = [plsc.load_gather(zv, [zoff + (1 + j)]) for j in range(D)]

        lo = jnp.zeros(16, jnp.int32)
        hi = jnp.full((16,), K, jnp.int32)
        for _step in range(11):
            mid = (lo + hi) // 2
            pm = plsc.load_gather(part_v, [jnp.minimum(mid, K - 1)])
            take_hi = pm <= u
            lo = jnp.where(take_hi, mid + 1, lo)
            hi = jnp.where(take_hi, hi, mid)
        idx = jnp.minimum(lo, K - 1)

        mbase = idx * MSTRIDE
        dbase = idx * DSTRIDE
        obase = rid * D
        for i in range(D):
            acc = plsc.load_gather(means_v, [mbase + i])
            for j in range(D):
                acc = acc + plsc.load_gather(devs_v, [dbase + (i * D + j)]) * x[j]
            plsc.store_scatter(outv, [obase + i], acc)


def _body(z_hbm, means_hbm, devs_hbm, part_hbm, out_hbm,
          devs_v, means_v, part_v, zv, outv):
    pltpu.sync_copy(devs_hbm, devs_v)
    pltpu.sync_copy(means_hbm, means_v)
    pltpu.sync_copy(part_hbm, part_v)

    wid = lax.axis_index("s") * 2 + lax.axis_index("c")

    def task(t, _):
        b = wid + NW * t

        @pl.when(b < NFULL)
        def _():
            zoff = pl.multiple_of(b * (R * (D + 1)), 8)
            ooff = pl.multiple_of(b * (R * D), 8)
            pltpu.sync_copy(z_hbm.at[pl.ds(zoff, R * (D + 1))], zv)
            _process_rows(zv, outv, devs_v, means_v, part_v, R // 16)
            pltpu.sync_copy(outv, out_hbm.at[pl.ds(ooff, R * D)])

        return 0

    lax.fori_loop(0, TASKS, task, 0)

    @pl.when(wid == TAIL_W)
    def _():
        zoff = pl.multiple_of(NFULL * (R * (D + 1)), 8)
        ooff = pl.multiple_of(NFULL * (R * D), 8)
        pltpu.sync_copy(z_hbm.at[pl.ds(zoff, TAIL * (D + 1))],
                        zv.at[pl.ds(0, TAIL * (D + 1))])
        _process_rows(zv, outv, devs_v, means_v, part_v, TAIL // 16)
        pltpu.sync_copy(outv.at[pl.ds(0, TAIL * D)],
                        out_hbm.at[pl.ds(ooff, TAIL * D)])


@jax.jit
def _run(zf, meansf, devsf, part):
    mesh = plsc.VectorSubcoreMesh(core_axis_name="c", subcore_axis_name="s")
    return pl.kernel(
        _body,
        mesh=mesh,
        compiler_params=pltpu.CompilerParams(needs_layout_passes=False),
        out_type=jax.ShapeDtypeStruct((N * D,), jnp.float32),
        scratch_types=[
            pltpu.VMEM((K * DSTRIDE,), jnp.float32),
            pltpu.VMEM((K * MSTRIDE,), jnp.float32),
            pltpu.VMEM((K,), jnp.float32),
            pltpu.VMEM((R * (D + 1),), jnp.float32),
            pltpu.VMEM((R * D,), jnp.float32),
        ],
    )(zf, meansf, devsf, part)


def kernel(z, means, devs, mix_partition):
    meansp = jnp.pad(means, ((0, 0), (0, MSTRIDE - D))).reshape(-1)
    devsp = jnp.pad(devs.reshape(K, D * D),
                    ((0, 0), (0, DSTRIDE - D * D))).reshape(-1)
    out = _run(z.reshape(-1), meansp, devsp, mix_partition)
    return out.reshape(N, D)
